# initial kernel scaffold (unmeasured)
import jax
import jax.numpy as jnp
from jax import lax
from jax.experimental import pallas as pl
from jax.experimental.pallas import tpu as pltpu

B, S, H, Dh, Dr = 2, 512, 16, 128, 32
D = 2048
DC = 128
N_X = 2
BS = B * S
SCALE = (Dh + Dr) ** -0.5
BF16 = jnp.bfloat16
F32 = jnp.float32

_VMEM = pl.BlockSpec(memory_space=pltpu.VMEM)


def _proj_body(x_ref, wq_ref, wqr_ref, wkr_ref, wdkv_ref,
               q_ref, qr_ref, kr_ref, c_ref):
    xb = x_ref[...].astype(BF16)
    q_ref[...] = jnp.dot(xb, wq_ref[...].astype(BF16),
                         preferred_element_type=F32).astype(BF16)
    qr_ref[...] = jnp.dot(xb, wqr_ref[...].astype(BF16),
                          preferred_element_type=F32).astype(BF16)
    kr_ref[...] = jnp.dot(xb, wkr_ref[...].astype(BF16),
                          preferred_element_type=F32).astype(BF16)
    c_ref[...] = jnp.dot(xb, wdkv_ref[...].astype(BF16),
                         preferred_element_type=F32).astype(BF16)


def _comm_body(c_ref, wuk_ref, wuv_ref,
               cg_ref, wukg_ref, wuvg_ref,
               send_sems, recv_sems):
    my_x = lax.axis_index("x")
    my_y = lax.axis_index("y")
    peer = (1 - my_x, my_y)

    barrier = pltpu.get_barrier_semaphore()
    pl.semaphore_signal(barrier, inc=1, device_id=peer,
                        device_id_type=pl.DeviceIdType.MESH)
    pl.semaphore_wait(barrier, 1)

    cg_ref[my_x] = c_ref[...]
    wukg_ref[my_x] = wuk_ref[...].astype(BF16)
    wuvg_ref[my_x] = wuv_ref[...].astype(BF16)

    rdmas = []
    for i, ref in enumerate((cg_ref, wukg_ref, wuvg_ref)):
        rdma = pltpu.make_async_remote_copy(
            src_ref=ref.at[my_x],
            dst_ref=ref.at[my_x],
            send_sem=send_sems.at[i],
            recv_sem=recv_sems.at[i],
            device_id=peer,
            device_id_type=pl.DeviceIdType.MESH,
        )
        rdma.start()
        rdmas.append(rdma)
    for rdma in rdmas:
        rdma.wait()


def _kv_body(cg_ref, wukg_ref, wuvg_ref, k_ref, v_ref):
    c0 = cg_ref[0]
    c1 = cg_ref[1]
    k_ref[...] = (jnp.dot(c0, wukg_ref[0], preferred_element_type=F32)
                  + jnp.dot(c1, wukg_ref[1],
                            preferred_element_type=F32)).astype(BF16)
    v_ref[...] = (jnp.dot(c0, wuvg_ref[0], preferred_element_type=F32)
                  + jnp.dot(c1, wuvg_ref[1],
                            preferred_element_type=F32)).astype(BF16)


def _attn_body(q_ref, k_ref, v_ref, qr_ref, kr_ref, o_ref):
    s = lax.dot_general(q_ref[...], k_ref[...], (((1,), (1,)), ((), ())),
                        preferred_element_type=F32)
    s += lax.dot_general(qr_ref[...], kr_ref[...], (((1,), (1,)), ((), ())),
                         preferred_element_type=F32)
    s *= SCALE
    m = jnp.max(s, axis=1, keepdims=True)
    e = jnp.exp(s - m)
    p = (e / jnp.sum(e, axis=1, keepdims=True)).astype(BF16)
    o_ref[...] = jnp.dot(p, v_ref[...], preferred_element_type=F32).astype(BF16)


def _out_body(o_ref, wo_ref, out_ref):
    out_ref[...] = jnp.dot(o_ref[...], wo_ref[...].astype(BF16),
                           preferred_element_type=F32)


def kernel(x, Wdkv, Wuk, Wuv, Wq, Wqr, Wkr, Wo):
    xb = x.reshape(BS, D)

    q, qr, kr, c_sh = pl.pallas_call(
        _proj_body,
        out_shape=(
            jax.ShapeDtypeStruct((BS, D), BF16),
            jax.ShapeDtypeStruct((BS, H * Dr), BF16),
            jax.ShapeDtypeStruct((BS, Dr), BF16),
            jax.ShapeDtypeStruct((BS, DC), BF16),
        ),
        in_specs=[_VMEM] * 5,
        out_specs=(_VMEM,) * 4,
    )(xb, Wq, Wqr, Wkr, Wdkv)

    cg, wukg, wuvg = pl.pallas_call(
        _comm_body,
        out_shape=(
            jax.ShapeDtypeStruct((N_X, BS, DC), BF16),
            jax.ShapeDtypeStruct((N_X, DC, D), BF16),
            jax.ShapeDtypeStruct((N_X, DC, D), BF16),
        ),
        in_specs=[_VMEM] * 3,
        out_specs=(_VMEM,) * 3,
        scratch_shapes=[
            pltpu.SemaphoreType.DMA((3,)),
            pltpu.SemaphoreType.DMA((3,)),
        ],
        compiler_params=pltpu.CompilerParams(collective_id=0),
    )(c_sh, Wuk, Wuv)

    k, v = pl.pallas_call(
        _kv_body,
        out_shape=(
            jax.ShapeDtypeStruct((BS, D), BF16),
            jax.ShapeDtypeStruct((BS, D), BF16),
        ),
        in_specs=[_VMEM] * 3,
        out_specs=(_VMEM,) * 2,
    )(cg, wukg, wuvg)

    o = pl.pallas_call(
        _attn_body,
        grid=(B, H),
        out_shape=jax.ShapeDtypeStruct((BS, D), BF16),
        in_specs=[
            pl.BlockSpec((S, Dh), lambda b, h: (b, h)),
            pl.BlockSpec((S, Dh), lambda b, h: (b, h)),
            pl.BlockSpec((S, Dh), lambda b, h: (b, h)),
            pl.BlockSpec((S, Dr), lambda b, h: (b, h)),
            pl.BlockSpec((S, Dr), lambda b, h: (b, 0)),
        ],
        out_specs=pl.BlockSpec((S, Dh), lambda b, h: (b, h)),
    )(q, k, v, qr, kr)

    out = pl.pallas_call(
        _out_body,
        out_shape=jax.ShapeDtypeStruct((BS, D), F32),
        in_specs=[_VMEM] * 2,
        out_specs=_VMEM,
    )(o, Wo)
    return out.reshape(B, S, D)


# baseline (device time: 90645 ns/iter reference)
import jax
import jax.numpy as jnp
from jax import lax
from jax.experimental import pallas as pl
from jax.experimental.pallas import tpu as pltpu

B, S, H, Dh, Dr = 2, 512, 16, 128, 32
D = 2048
DC = 128
N_X = 2
BS = B * S
SCALE = (Dh + Dr) ** -0.5
BF16 = jnp.bfloat16
F32 = jnp.float32

_VMEM = pl.BlockSpec(memory_space=pltpu.VMEM)


def _proj_body(x_ref, wq_ref, wqr_ref, wkr_ref, wdkv_ref,
               q_ref, qr_ref, kr_ref, c_ref):
    xb = x_ref[...].astype(BF16)
    q_ref[...] = jnp.dot(xb, wq_ref[...].astype(BF16),
                         preferred_element_type=F32).astype(BF16)
    qr_ref[...] = jnp.dot(xb, wqr_ref[...].astype(BF16),
                          preferred_element_type=F32).astype(BF16)
    kr_ref[...] = jnp.dot(xb, wkr_ref[...].astype(BF16),
                          preferred_element_type=F32).astype(BF16)
    c_ref[...] = jnp.dot(xb, wdkv_ref[...].astype(BF16),
                         preferred_element_type=F32).astype(BF16)


def _comm_body(c_ref, wuk_ref, wuv_ref,
               cg_ref, wukg_ref, wuvg_ref,
               send_sems, recv_sems):
    my_x = lax.axis_index("x")
    my_y = lax.axis_index("y")
    peer = (1 - my_x, my_y)

    barrier = pltpu.get_barrier_semaphore()
    pl.semaphore_signal(barrier, inc=1, device_id=peer,
                        device_id_type=pl.DeviceIdType.MESH)
    pl.semaphore_wait(barrier, 1)

    cg_ref[my_x] = c_ref[...]
    wukg_ref[my_x] = wuk_ref[...].astype(BF16)
    wuvg_ref[my_x] = wuv_ref[...].astype(BF16)

    rdmas = []
    for i, ref in enumerate((cg_ref, wukg_ref, wuvg_ref)):
        rdma = pltpu.make_async_remote_copy(
            src_ref=ref.at[my_x],
            dst_ref=ref.at[my_x],
            send_sem=send_sems.at[i],
            recv_sem=recv_sems.at[i],
            device_id=peer,
            device_id_type=pl.DeviceIdType.MESH,
        )
        rdma.start()
        rdmas.append(rdma)
    for rdma in rdmas:
        rdma.wait()


def _kv_body(cg_ref, wukg_ref, wuvg_ref, k_ref, v_ref):
    c0 = cg_ref[0]
    c1 = cg_ref[1]
    k_ref[...] = (jnp.dot(c0, wukg_ref[0], preferred_element_type=F32)
                  + jnp.dot(c1, wukg_ref[1],
                            preferred_element_type=F32)).astype(BF16)
    v_ref[...] = (jnp.dot(c0, wuvg_ref[0], preferred_element_type=F32)
                  + jnp.dot(c1, wuvg_ref[1],
                            preferred_element_type=F32)).astype(BF16)


def _attn_body(q_ref, k_ref, v_ref, qr_ref, kr_ref, o_ref):
    kr = kr_ref[...]
    for h in range(H):
        q = q_ref[:, h * Dh:(h + 1) * Dh]
        k = k_ref[:, h * Dh:(h + 1) * Dh]
        qr = qr_ref[:, h * Dr:(h + 1) * Dr]
        s = lax.dot_general(q, k, (((1,), (1,)), ((), ())),
                            preferred_element_type=F32)
        s += lax.dot_general(qr, kr, (((1,), (1,)), ((), ())),
                             preferred_element_type=F32)
        s *= SCALE
        m = jnp.max(s, axis=1, keepdims=True)
        e = jnp.exp(s - m)
        p = (e / jnp.sum(e, axis=1, keepdims=True)).astype(BF16)
        o_ref[:, h * Dh:(h + 1) * Dh] = jnp.dot(
            p, v_ref[:, h * Dh:(h + 1) * Dh],
            preferred_element_type=F32).astype(BF16)


def _out_body(o_ref, wo_ref, out_ref):
    out_ref[...] = jnp.dot(o_ref[...], wo_ref[...].astype(BF16),
                           preferred_element_type=F32)


def kernel(x, Wdkv, Wuk, Wuv, Wq, Wqr, Wkr, Wo):
    xb = x.reshape(BS, D)

    q, qr, kr, c_sh = pl.pallas_call(
        _proj_body,
        out_shape=(
            jax.ShapeDtypeStruct((BS, D), BF16),
            jax.ShapeDtypeStruct((BS, H * Dr), BF16),
            jax.ShapeDtypeStruct((BS, Dr), BF16),
            jax.ShapeDtypeStruct((BS, DC), BF16),
        ),
        in_specs=[_VMEM] * 5,
        out_specs=(_VMEM,) * 4,
    )(xb, Wq, Wqr, Wkr, Wdkv)

    cg, wukg, wuvg = pl.pallas_call(
        _comm_body,
        out_shape=(
            jax.ShapeDtypeStruct((N_X, BS, DC), BF16),
            jax.ShapeDtypeStruct((N_X, DC, D), BF16),
            jax.ShapeDtypeStruct((N_X, DC, D), BF16),
        ),
        in_specs=[_VMEM] * 3,
        out_specs=(_VMEM,) * 3,
        scratch_shapes=[
            pltpu.SemaphoreType.DMA((3,)),
            pltpu.SemaphoreType.DMA((3,)),
        ],
        compiler_params=pltpu.CompilerParams(collective_id=0),
    )(c_sh, Wuk, Wuv)

    k, v = pl.pallas_call(
        _kv_body,
        out_shape=(
            jax.ShapeDtypeStruct((BS, D), BF16),
            jax.ShapeDtypeStruct((BS, D), BF16),
        ),
        in_specs=[_VMEM] * 3,
        out_specs=(_VMEM,) * 2,
    )(cg, wukg, wuvg)

    o = pl.pallas_call(
        _attn_body,
        grid=(B,),
        out_shape=jax.ShapeDtypeStruct((BS, D), BF16),
        in_specs=[
            pl.BlockSpec((S, D), lambda b: (b, 0)),
            pl.BlockSpec((S, D), lambda b: (b, 0)),
            pl.BlockSpec((S, D), lambda b: (b, 0)),
            pl.BlockSpec((S, H * Dr), lambda b: (b, 0)),
            pl.BlockSpec((S, Dr), lambda b: (b, 0)),
        ],
        out_specs=pl.BlockSpec((S, D), lambda b: (b, 0)),
    )(q, k, v, qr, kr)

    out = pl.pallas_call(
        _out_body,
        out_shape=jax.ShapeDtypeStruct((BS, D), F32),
        in_specs=[_VMEM] * 2,
        out_specs=_VMEM,
    )(o, Wo)
    return out.reshape(B, S, D)


# device time: 67478 ns/iter; 1.3433x vs baseline; 1.3433x over previous
import jax
import jax.numpy as jnp
from jax import lax
from jax.experimental import pallas as pl
from jax.experimental.pallas import tpu as pltpu

B, S, H, Dh, Dr = 2, 512, 16, 128, 32
D = 2048
DC = 128
N_X = 2
N_DEV = 4
HG = H // N_DEV
GW = HG * Dh
GWR = HG * Dr
BS = B * S
SCALE = (Dh + Dr) ** -0.5
BF16 = jnp.bfloat16
F32 = jnp.float32

_VMEM = pl.BlockSpec(memory_space=pltpu.VMEM)
_MESH = pl.DeviceIdType.MESH


def _proj_body(x_ref, wq_ref, wqr_ref, wkr_ref, wdkv_ref,
               wuk_mine_ref, wuk_send_ref, wuv_mine_ref, wuv_send_ref,
               q_ref, qr_ref, kr_ref, kg_ref, vg_ref,
               cg, wukg, wuvg, wuk_sb, wuv_sb, send_sems, recv_sems):
    my_x = lax.axis_index("x")
    my_y = lax.axis_index("y")
    xpeer = (1 - my_x, my_y)

    barrier = pltpu.get_barrier_semaphore()
    pl.semaphore_signal(barrier, inc=1, device_id=xpeer, device_id_type=_MESH)
    pl.semaphore_wait(barrier, 1)

    xb = x_ref[...].astype(BF16)
    cg[my_x] = jnp.dot(xb, wdkv_ref[...].astype(BF16),
                       preferred_element_type=F32).astype(BF16)
    wukg[my_x] = wuk_mine_ref[...].astype(BF16)
    wuvg[my_x] = wuv_mine_ref[...].astype(BF16)
    wuk_sb[...] = wuk_send_ref[...].astype(BF16)
    wuv_sb[...] = wuv_send_ref[...].astype(BF16)

    rdmas = []
    for i, (src, dst) in enumerate((
            (cg.at[my_x], cg.at[my_x]),
            (wuk_sb, wukg.at[my_x]),
            (wuv_sb, wuvg.at[my_x]),
    )):
        rdma = pltpu.make_async_remote_copy(
            src_ref=src, dst_ref=dst,
            send_sem=send_sems.at[i], recv_sem=recv_sems.at[i],
            device_id=xpeer, device_id_type=_MESH,
        )
        rdma.start()
        rdmas.append(rdma)

    q_ref[...] = jnp.dot(xb, wq_ref[...].astype(BF16),
                         preferred_element_type=F32).astype(BF16)
    qr_ref[...] = jnp.dot(xb, wqr_ref[...].astype(BF16),
                          preferred_element_type=F32).astype(BF16)
    kr_ref[...] = jnp.dot(xb, wkr_ref[...].astype(BF16),
                          preferred_element_type=F32).astype(BF16)

    for rdma in rdmas:
        rdma.wait()

    kg_ref[...] = (jnp.dot(cg[0], wukg[0], preferred_element_type=F32)
                   + jnp.dot(cg[1], wukg[1],
                             preferred_element_type=F32)).astype(BF16)
    vg_ref[...] = (jnp.dot(cg[0], wuvg[0], preferred_element_type=F32)
                   + jnp.dot(cg[1], wuvg[1],
                             preferred_element_type=F32)).astype(BF16)


def _attn_body(q_ref, k_ref, v_ref, qr_ref, kr_ref, o_ref):
    kr = kr_ref[...]
    for h in range(HG):
        q = q_ref[:, h * Dh:(h + 1) * Dh]
        k = k_ref[:, h * Dh:(h + 1) * Dh]
        qr = qr_ref[:, h * Dr:(h + 1) * Dr]
        s = lax.dot_general(q, k, (((1,), (1,)), ((), ())),
                            preferred_element_type=F32)
        s += lax.dot_general(qr, kr, (((1,), (1,)), ((), ())),
                             preferred_element_type=F32)
        s *= SCALE
        m = jnp.max(s, axis=1, keepdims=True)
        e = jnp.exp(s - m)
        p = (e / jnp.sum(e, axis=1, keepdims=True)).astype(BF16)
        o_ref[:, h * Dh:(h + 1) * Dh] = jnp.dot(
            p, v_ref[:, h * Dh:(h + 1) * Dh],
            preferred_element_type=F32).astype(BF16)


def _gather_out_body(og_in_ref, wo_ref, out_ref, og, send_sems, recv_sems):
    my_x = lax.axis_index("x")
    my_y = lax.axis_index("y")
    g = 2 * my_x + my_y
    gx = 2 * (1 - my_x) + my_y
    gy = 2 * my_x + (1 - my_y)
    gd = 2 * (1 - my_x) + (1 - my_y)
    peers = ((1 - my_x, my_y), (my_x, 1 - my_y), (1 - my_x, 1 - my_y))

    barrier = pltpu.get_barrier_semaphore()
    for p_id in peers:
        pl.semaphore_signal(barrier, inc=1, device_id=p_id,
                            device_id_type=_MESH)
    pl.semaphore_wait(barrier, 3)

    og[g] = og_in_ref[...]
    rdmas = []
    for i, p_id in enumerate(peers):
        rdma = pltpu.make_async_remote_copy(
            src_ref=og.at[g], dst_ref=og.at[g],
            send_sem=send_sems.at[i], recv_sem=recv_sems.at[i],
            device_id=p_id, device_id_type=_MESH,
        )
        rdma.start()
        rdmas.append(rdma)

    def wo_slice(q):
        return wo_ref[pl.ds(q * GW, GW), :].astype(BF16)

    out_ref[...] = jnp.dot(og[g], wo_slice(g), preferred_element_type=F32)
    for rdma, slot in zip(rdmas, (gx, gy, gd)):
        rdma.wait_recv()
        out_ref[...] += jnp.dot(og[slot], wo_slice(slot),
                                preferred_element_type=F32)
    for rdma in rdmas:
        rdma.wait_send()


def kernel(x, Wdkv, Wuk, Wuv, Wq, Wqr, Wkr, Wo):
    xb = x.reshape(BS, D)
    gx_ = lax.axis_index("x")
    gy_ = lax.axis_index("y")
    g = 2 * gx_ + gy_
    peer_g = 2 * (1 - gx_) + gy_

    wq_g = lax.dynamic_slice(Wq, (0, g * GW), (D, GW))
    wqr_g = lax.dynamic_slice(Wqr, (0, g * GWR), (D, GWR))
    wuk_mine = lax.dynamic_slice(Wuk, (0, g * GW), (DC, GW))
    wuk_send = lax.dynamic_slice(Wuk, (0, peer_g * GW), (DC, GW))
    wuv_mine = lax.dynamic_slice(Wuv, (0, g * GW), (DC, GW))
    wuv_send = lax.dynamic_slice(Wuv, (0, peer_g * GW), (DC, GW))

    q, qr, kr, kg, vg = pl.pallas_call(
        _proj_body,
        out_shape=(
            jax.ShapeDtypeStruct((BS, GW), BF16),
            jax.ShapeDtypeStruct((BS, GWR), BF16),
            jax.ShapeDtypeStruct((BS, Dr), BF16),
            jax.ShapeDtypeStruct((BS, GW), BF16),
            jax.ShapeDtypeStruct((BS, GW), BF16),
        ),
        in_specs=[_VMEM] * 9,
        out_specs=(_VMEM,) * 5,
        scratch_shapes=[
            pltpu.VMEM((N_X, BS, DC), BF16),
            pltpu.VMEM((N_X, DC, GW), BF16),
            pltpu.VMEM((N_X, DC, GW), BF16),
            pltpu.VMEM((DC, GW), BF16),
            pltpu.VMEM((DC, GW), BF16),
            pltpu.SemaphoreType.DMA((3,)),
            pltpu.SemaphoreType.DMA((3,)),
        ],
        compiler_params=pltpu.CompilerParams(collective_id=0),
    )(xb, wq_g, wqr_g, Wkr, Wdkv, wuk_mine, wuk_send, wuv_mine, wuv_send)

    og = pl.pallas_call(
        _attn_body,
        grid=(B,),
        out_shape=jax.ShapeDtypeStruct((BS, GW), BF16),
        in_specs=[
            pl.BlockSpec((S, GW), lambda b: (b, 0)),
            pl.BlockSpec((S, GW), lambda b: (b, 0)),
            pl.BlockSpec((S, GW), lambda b: (b, 0)),
            pl.BlockSpec((S, GWR), lambda b: (b, 0)),
            pl.BlockSpec((S, Dr), lambda b: (b, 0)),
        ],
        out_specs=pl.BlockSpec((S, GW), lambda b: (b, 0)),
    )(q, kg, vg, qr, kr)

    out = pl.pallas_call(
        _gather_out_body,
        out_shape=jax.ShapeDtypeStruct((BS, D), F32),
        in_specs=[_VMEM] * 2,
        out_specs=_VMEM,
        scratch_shapes=[
            pltpu.VMEM((N_DEV, BS, GW), BF16),
            pltpu.SemaphoreType.DMA((3,)),
            pltpu.SemaphoreType.DMA((3,)),
        ],
        compiler_params=pltpu.CompilerParams(collective_id=1),
    )(og, Wo)
    return out.reshape(B, S, D)


# device time: 65375 ns/iter; 1.3865x vs baseline; 1.0322x over previous
import jax
import jax.numpy as jnp
from jax import lax
from jax.experimental import pallas as pl
from jax.experimental.pallas import tpu as pltpu

B, S, H, Dh, Dr = 2, 512, 16, 128, 32
D = 2048
DC = 128
N_X = 2
N_DEV = 4
HG = H // N_DEV
GW = HG * Dh
GWR = HG * Dr
BS = B * S
SCALE = (Dh + Dr) ** -0.5
BF16 = jnp.bfloat16
F32 = jnp.float32

_VMEM = pl.BlockSpec(memory_space=pltpu.VMEM)
_MESH = pl.DeviceIdType.MESH


def _proj_body(x_ref, wq_ref, wqr_ref, wkr_ref, wdkv_ref,
               wuk_mine_ref, wuk_send_ref, wuv_mine_ref, wuv_send_ref,
               q_ref, qr_ref, kr_ref, kg_ref, vg_ref,
               cg, wukg, wuvg, wuk_sb, wuv_sb, send_sems, recv_sems):
    my_x = lax.axis_index("x")
    my_y = lax.axis_index("y")
    xpeer = (1 - my_x, my_y)

    barrier = pltpu.get_barrier_semaphore()
    pl.semaphore_signal(barrier, inc=1, device_id=xpeer, device_id_type=_MESH)
    pl.semaphore_wait(barrier, 1)

    xb = x_ref[...].astype(BF16)
    cg[my_x] = jnp.dot(xb, wdkv_ref[...].astype(BF16),
                       preferred_element_type=F32).astype(BF16)
    wukg[my_x] = wuk_mine_ref[...].astype(BF16)
    wuvg[my_x] = wuv_mine_ref[...].astype(BF16)
    wuk_sb[...] = wuk_send_ref[...].astype(BF16)
    wuv_sb[...] = wuv_send_ref[...].astype(BF16)

    rdmas = []
    for i, (src, dst) in enumerate((
            (cg.at[my_x], cg.at[my_x]),
            (wuk_sb, wukg.at[my_x]),
            (wuv_sb, wuvg.at[my_x]),
    )):
        rdma = pltpu.make_async_remote_copy(
            src_ref=src, dst_ref=dst,
            send_sem=send_sems.at[i], recv_sem=recv_sems.at[i],
            device_id=xpeer, device_id_type=_MESH,
        )
        rdma.start()
        rdmas.append(rdma)

    q_ref[...] = jnp.dot(xb, wq_ref[...].astype(BF16),
                         preferred_element_type=F32).astype(BF16)
    qr_ref[...] = jnp.dot(xb, wqr_ref[...].astype(BF16),
                          preferred_element_type=F32).astype(BF16)
    kr_ref[...] = jnp.dot(xb, wkr_ref[...].astype(BF16),
                          preferred_element_type=F32).astype(BF16)

    for rdma in rdmas:
        rdma.wait()

    kg_ref[...] = (jnp.dot(cg[0], wukg[0], preferred_element_type=F32)
                   + jnp.dot(cg[1], wukg[1],
                             preferred_element_type=F32)).astype(BF16)
    vg_ref[...] = (jnp.dot(cg[0], wuvg[0], preferred_element_type=F32)
                   + jnp.dot(cg[1], wuvg[1],
                             preferred_element_type=F32)).astype(BF16)


def _attn_out_body(q_ref, k_ref, v_ref, qr_ref, kr_ref, wo_ref,
                   out_ref, og, send_sems, recv_sems):
    my_x = lax.axis_index("x")
    my_y = lax.axis_index("y")
    g = 2 * my_x + my_y
    gx = 2 * (1 - my_x) + my_y
    gy = 2 * my_x + (1 - my_y)
    gd = 2 * (1 - my_x) + (1 - my_y)
    peers = ((1 - my_x, my_y), (my_x, 1 - my_y), (1 - my_x, 1 - my_y))

    barrier = pltpu.get_barrier_semaphore()
    for p_id in peers:
        pl.semaphore_signal(barrier, inc=1, device_id=p_id,
                            device_id_type=_MESH)
    pl.semaphore_wait(barrier, 3)

    rdmas = []
    for h in range(HG):
        for b in range(B):
            rows = pl.ds(b * S, S)
            q = q_ref[rows, h * Dh:(h + 1) * Dh]
            k = k_ref[rows, h * Dh:(h + 1) * Dh]
            qr = qr_ref[rows, h * Dr:(h + 1) * Dr]
            s = lax.dot_general(q, k, (((1,), (1,)), ((), ())),
                                preferred_element_type=F32)
            s += lax.dot_general(qr, kr_ref[rows, :],
                                 (((1,), (1,)), ((), ())),
                                 preferred_element_type=F32)
            s *= SCALE
            m = jnp.max(s, axis=1, keepdims=True)
            e = jnp.exp(s - m)
            p = (e / jnp.sum(e, axis=1, keepdims=True)).astype(BF16)
            og[g, rows, h * Dh:(h + 1) * Dh] = jnp.dot(
                p, v_ref[rows, h * Dh:(h + 1) * Dh],
                preferred_element_type=F32).astype(BF16)
        for i, p_id in enumerate(peers):
            rdma = pltpu.make_async_remote_copy(
                src_ref=og.at[g, :, pl.ds(h * Dh, Dh)],
                dst_ref=og.at[g, :, pl.ds(h * Dh, Dh)],
                send_sem=send_sems.at[h * 3 + i],
                recv_sem=recv_sems.at[h * 3 + i],
                device_id=p_id, device_id_type=_MESH,
            )
            rdma.start()
            rdmas.append(rdma)

    def wo_slice(q_):
        return wo_ref[pl.ds(q_ * GW, GW), :].astype(BF16)

    out_ref[...] = jnp.dot(og[g], wo_slice(g), preferred_element_type=F32)
    for i, slot in enumerate((gx, gy, gd)):
        for h in range(HG):
            rdmas[h * 3 + i].wait_recv()
        out_ref[...] += jnp.dot(og[slot], wo_slice(slot),
                                preferred_element_type=F32)
    for rdma in rdmas:
        rdma.wait_send()


def kernel(x, Wdkv, Wuk, Wuv, Wq, Wqr, Wkr, Wo):
    xb = x.reshape(BS, D)
    gx_ = lax.axis_index("x")
    gy_ = lax.axis_index("y")
    g = 2 * gx_ + gy_
    peer_g = 2 * (1 - gx_) + gy_

    wq_g = lax.dynamic_slice(Wq, (0, g * GW), (D, GW))
    wqr_g = lax.dynamic_slice(Wqr, (0, g * GWR), (D, GWR))
    wuk_mine = lax.dynamic_slice(Wuk, (0, g * GW), (DC, GW))
    wuk_send = lax.dynamic_slice(Wuk, (0, peer_g * GW), (DC, GW))
    wuv_mine = lax.dynamic_slice(Wuv, (0, g * GW), (DC, GW))
    wuv_send = lax.dynamic_slice(Wuv, (0, peer_g * GW), (DC, GW))

    q, qr, kr, kg, vg = pl.pallas_call(
        _proj_body,
        out_shape=(
            jax.ShapeDtypeStruct((BS, GW), BF16),
            jax.ShapeDtypeStruct((BS, GWR), BF16),
            jax.ShapeDtypeStruct((BS, Dr), BF16),
            jax.ShapeDtypeStruct((BS, GW), BF16),
            jax.ShapeDtypeStruct((BS, GW), BF16),
        ),
        in_specs=[_VMEM] * 9,
        out_specs=(_VMEM,) * 5,
        scratch_shapes=[
            pltpu.VMEM((N_X, BS, DC), BF16),
            pltpu.VMEM((N_X, DC, GW), BF16),
            pltpu.VMEM((N_X, DC, GW), BF16),
            pltpu.VMEM((DC, GW), BF16),
            pltpu.VMEM((DC, GW), BF16),
            pltpu.SemaphoreType.DMA((3,)),
            pltpu.SemaphoreType.DMA((3,)),
        ],
        compiler_params=pltpu.CompilerParams(collective_id=0),
    )(xb, wq_g, wqr_g, Wkr, Wdkv, wuk_mine, wuk_send, wuv_mine, wuv_send)

    out = pl.pallas_call(
        _attn_out_body,
        out_shape=jax.ShapeDtypeStruct((BS, D), F32),
        in_specs=[_VMEM] * 6,
        out_specs=_VMEM,
        scratch_shapes=[
            pltpu.VMEM((N_DEV, BS, GW), BF16),
            pltpu.SemaphoreType.DMA((HG * 3,)),
            pltpu.SemaphoreType.DMA((HG * 3,)),
        ],
        compiler_params=pltpu.CompilerParams(collective_id=1),
    )(q, kg, vg, qr, kr, Wo)
    return out.reshape(B, S, D)
